# baseline (speedup 1.0000x reference)
import functools

import jax
import jax.numpy as jnp
from jax.experimental import pallas as pl
from jax.experimental.pallas import tpu as pltpu

N_NODES = 10000
N_EDGES = 320000
D = 128
B = 2560
W = 128
NB = N_EDGES // B


def _scatter_kernel(bounds_ref, ins_ref, ids_ref, w1_ref, b1_ref,
                    acc_ref, hc_buf):
    step = pl.program_id(0)

    @pl.when(step == 0)
    def _init():
        acc_ref[...] = jnp.zeros_like(acc_ref)

    lane = jax.lax.broadcasted_iota(jnp.int32, (B, W), 1)

    p = jax.lax.rem(step, 2)
    x = ins_ref[...].astype(jnp.bfloat16)
    h = jnp.maximum(
        jnp.dot(x, w1_ref[...], preferred_element_type=jnp.float32)
        + b1_ref[...],
        0.0,
    ).astype(jnp.bfloat16)
    hc_buf[p, :, :D] = h
    hc_buf[p, :, D:] = (lane == 0).astype(jnp.bfloat16)

    ids = ids_ref[0]
    lo = bounds_ref[0, step]
    hi = bounds_ref[1, step]
    hc = hc_buf[1 - p]

    def pass_body(pp):
        base = pl.multiple_of(pp, 8)
        rel = ids - base
        oh = (rel == lane).astype(jnp.bfloat16)
        seg = jax.lax.dot_general(
            oh, hc, (((0,), (0,)), ((), ())),
            preferred_element_type=jnp.float32,
        )
        acc_ref[pl.ds(base, W), :] += seg
        return base + W

    jax.lax.while_loop(lambda pp: pp <= hi, pass_body, (lo // 8) * 8)


def _finish_kernel(acc_ref, w2_ref, b2_ref,
                   rw1_ref, rb1_ref, rw2_ref, rb2_ref, out_ref):
    a = acc_ref[...]
    c = a[:, D:D + 1]
    g = a[:, :D] / jnp.maximum(c, 1.0)
    hm = (
        jnp.dot(g, w2_ref[...], preferred_element_type=jnp.float32)
        + b2_ref[...] * (c > 0)
    )
    h1 = jnp.maximum(
        jnp.dot(hm, rw1_ref[...], preferred_element_type=jnp.float32)
        + rb1_ref[...],
        0.0,
    )
    out_ref[...] = (
        jnp.dot(h1, rw2_ref[...], preferred_element_type=jnp.float32)
        + rb2_ref[...]
    )


@functools.partial(jax.jit, static_argnames=("interpret",))
def _run(ins, batch, phi_W1, phi_b1, phi_W2, phi_b2,
         rho_W1, rho_b1, rho_W2, rho_b2, interpret=False):
    ids = jnp.asarray(batch, jnp.int32)
    ids3 = ids.reshape(NB, B, 1)
    sent = jnp.array([[0], [-1]], jnp.int32)
    bounds = jnp.concatenate(
        [sent, jnp.stack([ids[0::B], ids[B - 1::B]])], axis=1)
    n_pad = N_NODES + W
    acc = pl.pallas_call(
        _scatter_kernel,
        grid_spec=pltpu.PrefetchScalarGridSpec(
            num_scalar_prefetch=1,
            grid=(NB + 1,),
            in_specs=[
                pl.BlockSpec((B, D), lambda i, s: (jnp.minimum(i, NB - 1), 0)),
                pl.BlockSpec((1, B, 1),
                             lambda i, s: (jnp.maximum(i - 1, 0), 0, 0)),
                pl.BlockSpec((D, D), lambda i, s: (0, 0)),
                pl.BlockSpec((1, D), lambda i, s: (0, 0)),
            ],
            out_specs=pl.BlockSpec((n_pad, D + W), lambda i, s: (0, 0)),
            scratch_shapes=[pltpu.VMEM((2, B, D + W), jnp.bfloat16)],
        ),
        out_shape=jax.ShapeDtypeStruct((n_pad, D + W), jnp.float32),
        compiler_params=pltpu.CompilerParams(
            dimension_semantics=("arbitrary",),
        ),
        interpret=interpret,
    )(bounds, ins, ids3, phi_W1.astype(jnp.bfloat16), phi_b1.reshape(1, D))

    R = 1000
    out = pl.pallas_call(
        _finish_kernel,
        grid=(N_NODES // R,),
        in_specs=[
            pl.BlockSpec((R, D + W), lambda i: (i, 0)),
            pl.BlockSpec((D, D), lambda i: (0, 0)),
            pl.BlockSpec((1, D), lambda i: (0, 0)),
            pl.BlockSpec((D, D), lambda i: (0, 0)),
            pl.BlockSpec((1, D), lambda i: (0, 0)),
            pl.BlockSpec((D, D), lambda i: (0, 0)),
            pl.BlockSpec((1, D), lambda i: (0, 0)),
        ],
        out_specs=pl.BlockSpec((R, D), lambda i: (i, 0)),
        out_shape=jax.ShapeDtypeStruct((N_NODES, D), jnp.float32),
        interpret=interpret,
    )(acc, phi_W2, phi_b2.reshape(1, D),
      rho_W1, rho_b1.reshape(1, D), rho_W2, rho_b2.reshape(1, D))
    return out


def kernel(ins, batch, dim, phi_W1, phi_b1, phi_W2, phi_b2,
           rho_W1, rho_b1, rho_W2, rho_b2):
    return _run(ins, batch, phi_W1, phi_b1, phi_W2, phi_b2,
                rho_W1, rho_b1, rho_W2, rho_b2)
